# baseline (speedup 1.0000x reference)
import functools

import jax
import jax.numpy as jnp
from jax import lax
from jax.experimental import pallas as pl
from jax.experimental.pallas import tpu as pltpu
from jax.experimental.pallas import tpu_sc as plsc

D_MODEL = 64
SCALE = 8.0
NUM_CORES = 2
NUM_SUBCORES = 16
NUM_WORKERS = NUM_CORES * NUM_SUBCORES
LANES = 16
NB = 4
UT = NB * 128

B = 4096
L = 200
BT = B // 128
N_UNITS = L * (BT // NB)
UNITS_PER_W = N_UNITS // NUM_WORKERS
UPL = BT // NB


@jax.jit
def _emb_lookup(idx_flat, table):
    mesh = plsc.VectorSubcoreMesh(core_axis_name="c", subcore_axis_name="s")

    @functools.partial(
        pl.kernel,
        mesh=mesh,
        out_type=jax.ShapeDtypeStruct((L, D_MODEL // 8, BT, 8, 128), jnp.float32),
        scratch_types=[
            pltpu.VMEM((UNITS_PER_W * UT,), jnp.int32),
            [pltpu.VMEM((UT, D_MODEL), jnp.float32) for _ in range(2)],
            pltpu.VMEM((D_MODEL // 8, NB, 8, 128), jnp.float32),
            [pltpu.SemaphoreType.DMA for _ in range(2)],
            pltpu.SemaphoreType.DMA,
        ],
        compiler_params=pltpu.CompilerParams(
            use_tc_tiling_on_sc=False, needs_layout_passes=False
        ),
    )
    def k(idx_hbm, table_hbm, out_hbm, idx_v, gbufs, sb, sem_g, sem_w):
        wid = lax.axis_index("s") * NUM_CORES + lax.axis_index("c")
        base_tok = wid * UNITS_PER_W * UT
        pltpu.sync_copy(idx_hbm.at[pl.ds(base_tok, UNITS_PER_W * UT)], idx_v)
        iota = lax.iota(jnp.int32, LANES)

        def unit_lbt(g):
            u = wid * UNITS_PER_W + g
            return u // UPL, (u % UPL) * NB

        def fire_gather(g, b):
            pltpu.async_copy(
                table_hbm.at[idx_v.at[pl.ds(g * UT, UT)]], gbufs[b], sem_g[b]
            )

        def wait_gather(g, b):
            pltpu.make_async_copy(
                table_hbm.at[idx_v.at[pl.ds(g * UT, UT)]], gbufs[b], sem_g[b]
            ).wait()

        def fire_write(g):
            l, bt0 = unit_lbt(g)
            pltpu.async_copy(sb, out_hbm.at[l, :, pl.ds(bt0, NB)], sem_w)

        def wait_write(g):
            l, bt0 = unit_lbt(g)
            pltpu.make_async_copy(
                sb, out_hbm.at[l, :, pl.ds(bt0, NB)], sem_w
            ).wait()

        fire_gather(0, 0)

        def unit_body(r, carry):
            for j in range(2):
                g = r * 2 + j
                wait_gather(g, j)

                @pl.when(g + 1 < UNITS_PER_W)
                def _():
                    fire_gather(g + 1, 1 - j)

                @pl.when(g >= 1)
                def _():
                    wait_write(g - 1)

                gb = gbufs[j]

                @plsc.parallel_loop(0, D_MODEL, 1, unroll=2)
                def _transp(d):
                    dt = d // 8
                    dr = d % 8
                    col = jnp.full((LANES,), d, jnp.int32)
                    for tg in range(UT // LANES):
                        v = plsc.load_gather(gb, [iota + (tg * LANES), col])
                        sb[dt, tg // 8, dr, pl.ds((tg % 8) * LANES, LANES)] = (
                            v * SCALE
                        )

                fire_write(g)

            return carry

        lax.fori_loop(0, UNITS_PER_W // 2, unit_body, 0)
        wait_write(UNITS_PER_W - 1)

    return k(idx_flat, table)


def kernel(x, table):
    b, l = x.shape
    idx_flat = x.T.reshape(b * l).astype(jnp.int32)
    raw = _emb_lookup(idx_flat, table)
    return raw.transpose(2, 4, 0, 1, 3).reshape(b, l, D_MODEL)


# speedup vs baseline: 1.2917x; 1.2917x over previous
import functools

import jax
import jax.numpy as jnp
from jax import lax
from jax.experimental import pallas as pl
from jax.experimental.pallas import tpu as pltpu
from jax.experimental.pallas import tpu_sc as plsc

D_MODEL = 64
PADW = 128
SCALE = 8.0
NUM_CORES = 2
NUM_SUBCORES = 16
NUM_WORKERS = NUM_CORES * NUM_SUBCORES
LANES = 16
CHUNK = 256

B = 4096
L = 200
N_ROWS = B * L
ROWS_PER_W = N_ROWS // NUM_WORKERS
N_CHUNKS = ROWS_PER_W // CHUNK


@jax.jit
def _emb_lookup(idx_flat, table):
    mesh = plsc.VectorSubcoreMesh(core_axis_name="c", subcore_axis_name="s")

    @functools.partial(
        pl.kernel,
        mesh=mesh,
        out_type=jax.ShapeDtypeStruct((N_ROWS, PADW), jnp.float32),
        scratch_types=[
            pltpu.VMEM((ROWS_PER_W,), jnp.int32),
            [pltpu.VMEM((CHUNK, D_MODEL), jnp.float32) for _ in range(2)],
            [pltpu.VMEM((CHUNK, PADW), jnp.float32) for _ in range(2)],
            [pltpu.SemaphoreType.DMA for _ in range(2)],
            [pltpu.SemaphoreType.DMA for _ in range(2)],
        ],
        compiler_params=pltpu.CompilerParams(
            use_tc_tiling_on_sc=False, needs_layout_passes=False
        ),
    )
    def k(idx_hbm, table_hbm, out_hbm, idx_v, gbufs, sbufs, sem_g, sem_w):
        wid = lax.axis_index("s") * NUM_CORES + lax.axis_index("c")
        base = wid * ROWS_PER_W
        pltpu.sync_copy(idx_hbm.at[pl.ds(base, ROWS_PER_W)], idx_v)

        def fire_gather(g, b):
            pltpu.async_copy(
                table_hbm.at[idx_v.at[pl.ds(g * CHUNK, CHUNK)]], gbufs[b], sem_g[b]
            )

        def wait_gather(g, b):
            pltpu.make_async_copy(
                table_hbm.at[idx_v.at[pl.ds(g * CHUNK, CHUNK)]], gbufs[b], sem_g[b]
            ).wait()

        def fire_write(g, b):
            pltpu.async_copy(
                sbufs[b], out_hbm.at[pl.ds(base + g * CHUNK, CHUNK)], sem_w[b]
            )

        def wait_write(g, b):
            pltpu.make_async_copy(
                sbufs[b], out_hbm.at[pl.ds(base + g * CHUNK, CHUNK)], sem_w[b]
            ).wait()

        fire_gather(0, 0)

        def round_body(r, carry):
            for j in range(2):
                g = r * 2 + j
                wait_gather(g, j)

                @pl.when(g + 1 < N_CHUNKS)
                def _():
                    fire_gather(g + 1, 1 - j)

                @pl.when(g >= 2)
                def _():
                    wait_write(g - 2, j)

                gb, sb = gbufs[j], sbufs[j]

                @plsc.parallel_loop(0, CHUNK, 1, unroll=4)
                def _scale(row):
                    for kk in range(D_MODEL // LANES):
                        sl = pl.ds(kk * LANES, LANES)
                        sb[row, sl] = gb[row, sl] * SCALE

                fire_write(g, j)

            return carry

        lax.fori_loop(0, N_CHUNKS // 2, round_body, 0)
        wait_write(N_CHUNKS - 2, 0)
        wait_write(N_CHUNKS - 1, 1)

    return k(idx_flat, table)


def kernel(x, table):
    b, l = x.shape
    idx_flat = x.reshape(b * l).astype(jnp.int32)
    raw = _emb_lookup(idx_flat, table)
    return raw.reshape(b, l, PADW)[:, :, :D_MODEL]
